# baseline (speedup 1.0000x reference)
import functools

import jax
import jax.numpy as jnp
from jax import lax
from jax.experimental import pallas as pl
from jax.experimental.pallas import tpu as pltpu
from jax.experimental.pallas import tpu_sc as plsc

N_NODES = 10000
IN_DIM = 128
N_EDGES = 320000
NC = 2
NS = 16
NW = NC * NS
E_PER_W = N_EDGES // NW
E_CHK = 80
N_CHK_REAL = E_PER_W // E_CHK
N_CHK = 126
N_GRP = N_CHK // 3
N_SGRP = N_GRP // 3
IO_ROWS = 80
N_IO_BLKS = N_NODES // IO_ROWS

_SPLAT_DN = lax.GatherDimensionNumbers(
    offset_dims=(), collapsed_slice_dims=(0,), start_index_map=(0,))


def _splat(v16, j):
  idx = jnp.full((16, 1), j, dtype=jnp.int32)
  return lax.gather(v16, idx, _SPLAT_DN, (1,),
                    mode=lax.GatherScatterMode.PROMISE_IN_BOUNDS)


NBUF = 3


def _sc_body(x_hbm, ep_hbm, out_hbm, outd_hbm,
             acc_sh, ep0, ep1, ep2, rows0, rows1, rows2, deg_v,
             e0, e1, e2, g0, g1, g2, s0, s1, s2):
  c = lax.axis_index("c")
  s = lax.axis_index("s")
  wid = c * NS + s
  eps = (ep0, ep1, ep2)
  rows = (rows0, rows1, rows2)
  esem = (e0, e1, e2)
  gsem = (g0, g1, g2)
  ssem = (s0, s1, s2)

  def _wait_gather(b):
    pltpu.make_async_copy(x_hbm.at[pl.ds(0, E_CHK)], rows[b],
                          gsem[b]).wait()

  def _wait_scatter(b):
    pltpu.make_async_copy(rows[b], acc_sh.at[pl.ds(0, E_CHK)],
                          ssem[b]).wait()

  def _copy_group(grp, eb):
    pltpu.async_copy(ep_hbm.at[wid, grp], eps[eb], esem[eb])

  def _wait_group(eb):
    pltpu.make_async_copy(ep_hbm.at[wid, 0], eps[eb], esem[eb]).wait()

  def _zero_row(i, _):
    for f in range(IN_DIM // 16):
      rows0[i, pl.ds(f * 16, 16)] = jnp.zeros((16,), jnp.float32)
    return 0
  lax.fori_loop(0, IO_ROWS, _zero_row, 0)

  def _zero_deg(i, _):
    deg_v[pl.ds(i * 16, 16)] = jnp.zeros((16,), jnp.float32)
    return 0
  lax.fori_loop(0, N_NODES // 16, _zero_deg, 0)

  def _zero_acc(t, _):
    blk = t * NS + s
    @pl.when(blk < N_IO_BLKS)
    def _():
      pltpu.sync_copy(rows0, acc_sh.at[pl.ds(blk * IO_ROWS, IO_ROWS)])
    return 0
  lax.fori_loop(0, (N_IO_BLKS + NS - 1) // NS, _zero_acc, 0)

  plsc.subcore_barrier()

  def _do_chunk(j, g2, tb):
    b = tb
    nb = (tb + 1) % NBUF
    eb = g2
    neb = (g2 + (1 if tb == 2 else 0)) % 3

    @pl.when(j + 1 < N_CHK)
    def _():
      @pl.when(j + 1 >= NBUF)
      def _():
        _wait_scatter(nb)
      if tb == 1:
        grp = (j + 6) // 3
        @pl.when(grp < N_GRP)
        def _():
          _copy_group(grp, (g2 + 2) % 3)
      if tb == 2:
        _wait_group(neb)
      ntb = (tb + 1) % 3
      pltpu.async_copy(x_hbm.at[eps[neb].at[3 * ntb]], rows[nb], gsem[nb])

    _wait_gather(b)

    def _scale16(k, _):
      w16 = plsc.bitcast(eps[eb][3 * tb + 2, pl.ds(k * 16, 16)], jnp.float32)
      c16 = eps[eb][3 * tb + 1, pl.ds(k * 16, 16)]
      plsc.addupdate_scatter(deg_v, [c16], w16)
      for j2 in range(16):
        e = k * 16 + j2
        ws = _splat(w16, j2)
        for f in range(IN_DIM // 16):
          sl = pl.ds(f * 16, 16)
          rows[b][e, sl] = rows[b][e, sl] * ws
      return 0
    lax.fori_loop(0, E_CHK // 16, _scale16, 0)

    pltpu.async_copy(rows[b], acc_sh.at[eps[eb].at[3 * tb + 1]], ssem[b],
                     add=True)

  _copy_group(0, 0)
  _copy_group(1, 1)
  _wait_group(0)
  pltpu.async_copy(x_hbm.at[eps[0].at[0]], rows[0], gsem[0])

  def _super(sg, _):
    jbase = sg * 9
    for g2 in range(3):
      for tb in range(3):
        _do_chunk(jbase + 3 * g2 + tb, g2, tb)
    return 0
  lax.fori_loop(0, N_SGRP, _super, 0)

  for b in range(NBUF):
    _wait_scatter(b)

  pltpu.sync_copy(deg_v, outd_hbm.at[wid])

  plsc.subcore_barrier()

  def _writeout(t, _):
    blk = t * NS + s
    @pl.when(blk < N_IO_BLKS)
    def _():
      start = blk * IO_ROWS
      pltpu.sync_copy(acc_sh.at[pl.ds(start, IO_ROWS)], rows0)
      pltpu.sync_copy(rows0, out_hbm.at[c, pl.ds(start, IO_ROWS)])
    return 0
  lax.fori_loop(0, (N_IO_BLKS + NS - 1) // NS, _writeout, 0)


def _sc_aggregate(x, epack):
  mesh = plsc.VectorSubcoreMesh(core_axis_name="c", subcore_axis_name="s")
  return pl.kernel(
      _sc_body,
      out_type=(
          jax.ShapeDtypeStruct((NC, N_NODES, IN_DIM), jnp.float32),
          jax.ShapeDtypeStruct((NW, N_NODES), jnp.float32),
      ),
      mesh=mesh,
      compiler_params=pltpu.CompilerParams(needs_layout_passes=False),
      scratch_types=[
          pltpu.VMEM_SHARED((N_NODES, IN_DIM), jnp.float32),
          pltpu.VMEM((9, E_CHK), jnp.int32),
          pltpu.VMEM((9, E_CHK), jnp.int32),
          pltpu.VMEM((9, E_CHK), jnp.int32),
          pltpu.VMEM((E_CHK, IN_DIM), jnp.float32),
          pltpu.VMEM((E_CHK, IN_DIM), jnp.float32),
          pltpu.VMEM((E_CHK, IN_DIM), jnp.float32),
          pltpu.VMEM((N_NODES,), jnp.float32),
          pltpu.SemaphoreType.DMA,
          pltpu.SemaphoreType.DMA,
          pltpu.SemaphoreType.DMA,
          pltpu.SemaphoreType.DMA,
          pltpu.SemaphoreType.DMA,
          pltpu.SemaphoreType.DMA,
          pltpu.SemaphoreType.DMA,
          pltpu.SemaphoreType.DMA,
          pltpu.SemaphoreType.DMA,
      ],
  )(x, epack)


ROW_BLK = 1000


def _tc_body(x_ref, acc_ref, deg_ref, wsT_ref, wnT_ref, bs_ref, bn_ref,
             out_ref):
  neigh = acc_ref[0] + acc_ref[1]
  deg = jnp.sum(deg_ref[0], axis=1, keepdims=True)
  xn = neigh / jnp.maximum(deg, 1.0)
  out_ref[...] = (
      jnp.dot(x_ref[...], wsT_ref[...], preferred_element_type=jnp.float32)
      + jnp.dot(xn, wnT_ref[...], preferred_element_type=jnp.float32)
      + bs_ref[...] + bn_ref[...])


def _tc_combine(x, acc, degT, W_self, b_self, W_neigh, b_neigh):
  grid = (N_NODES // ROW_BLK,)
  return pl.pallas_call(
      _tc_body,
      grid=grid,
      in_specs=[
          pl.BlockSpec((ROW_BLK, IN_DIM), lambda i: (i, 0)),
          pl.BlockSpec((NC, ROW_BLK, IN_DIM), lambda i: (0, i, 0)),
          pl.BlockSpec((1, ROW_BLK, NW), lambda i: (i, 0, 0)),
          pl.BlockSpec((IN_DIM, IN_DIM), lambda i: (0, 0)),
          pl.BlockSpec((IN_DIM, IN_DIM), lambda i: (0, 0)),
          pl.BlockSpec((1, IN_DIM), lambda i: (0, 0)),
          pl.BlockSpec((1, IN_DIM), lambda i: (0, 0)),
      ],
      out_specs=pl.BlockSpec((ROW_BLK, IN_DIM), lambda i: (i, 0)),
      out_shape=jax.ShapeDtypeStruct((N_NODES, IN_DIM), jnp.float32),
  )(x, acc, degT, W_self.T, W_neigh.T, b_self[None, :], b_neigh[None, :])


@jax.jit
def kernel(x, edge_index, edge_weight, W_self, b_self, W_neigh, b_neigh):
  row = edge_index[0].astype(jnp.int32).reshape(NW, N_CHK_REAL, 1, E_CHK)
  col = edge_index[1].astype(jnp.int32).reshape(NW, N_CHK_REAL, 1, E_CHK)
  wbits = lax.bitcast_convert_type(edge_weight, jnp.int32).reshape(
      NW, N_CHK_REAL, 1, E_CHK)
  epack = jnp.concatenate([row, col, wbits], axis=2)
  epack = jnp.pad(epack, ((0, 0), (0, 1), (0, 0), (0, 0)))
  epack = epack.reshape(NW, N_GRP, 9, E_CHK)
  acc, deg = _sc_aggregate(x, epack)
  degT = deg.reshape(NW, N_NODES // ROW_BLK, ROW_BLK).transpose(1, 2, 0)
  return _tc_combine(x, acc, degT, W_self, b_self, W_neigh, b_neigh)


# speedup vs baseline: 1.2791x; 1.2791x over previous
import functools

import jax
import jax.numpy as jnp
from jax import lax
from jax.experimental import pallas as pl
from jax.experimental.pallas import tpu as pltpu
from jax.experimental.pallas import tpu_sc as plsc

N_NODES = 10000
IN_DIM = 128
N_EDGES = 320000
NC = 2
NS = 16
NW = NC * NS
E_PER_W = N_EDGES // NW
E_CHK = 80
N_CHK = E_PER_W // E_CHK
IO_ROWS = 80
N_IO_BLKS = N_NODES // IO_ROWS

_SPLAT_DN = lax.GatherDimensionNumbers(
    offset_dims=(), collapsed_slice_dims=(0,), start_index_map=(0,))


def _splat(v16, j):
  idx = jnp.full((16, 1), j, dtype=jnp.int32)
  return lax.gather(v16, idx, _SPLAT_DN, (1,),
                    mode=lax.GatherScatterMode.PROMISE_IN_BOUNDS)


NBUF = 3


def _sc_body(x_hbm, ep_hbm, out_hbm, outd_hbm,
             acc_sh, ep0, ep1, ep2, rows0, rows1, rows2, deg_v,
             g0, g1, g2, s0, s1, s2):
  c = lax.axis_index("c")
  s = lax.axis_index("s")
  wid = c * NS + s
  eps = (ep0, ep1, ep2)
  rows = (rows0, rows1, rows2)
  gsem = (g0, g1, g2)
  ssem = (s0, s1, s2)

  def _wait_gather(b):
    pltpu.make_async_copy(x_hbm.at[pl.ds(0, E_CHK)], rows[b],
                          gsem[b]).wait()

  def _wait_scatter(b):
    pltpu.make_async_copy(rows[b], acc_sh.at[pl.ds(0, E_CHK)],
                          ssem[b]).wait()

  def _start_chunk(j, b):
    pltpu.sync_copy(ep_hbm.at[wid, j], eps[b])
    pltpu.async_copy(x_hbm.at[eps[b].at[0]], rows[b], gsem[b])

  def _zero_row(i, _):
    for f in range(IN_DIM // 16):
      rows0[i, pl.ds(f * 16, 16)] = jnp.zeros((16,), jnp.float32)
    return 0
  lax.fori_loop(0, IO_ROWS, _zero_row, 0)

  def _zero_deg(i, _):
    deg_v[pl.ds(i * 16, 16)] = jnp.zeros((16,), jnp.float32)
    return 0
  lax.fori_loop(0, N_NODES // 16, _zero_deg, 0)

  def _zero_acc(t, _):
    blk = t * NS + s
    @pl.when(blk < N_IO_BLKS)
    def _():
      pltpu.sync_copy(rows0, acc_sh.at[pl.ds(blk * IO_ROWS, IO_ROWS)])
    return 0
  lax.fori_loop(0, (N_IO_BLKS + NS - 1) // NS, _zero_acc, 0)

  plsc.subcore_barrier()

  def _do_chunk(j, b):
    nb = (b + 1) % NBUF

    @pl.when(j + 1 < N_CHK)
    def _():
      @pl.when(j + 1 >= NBUF)
      def _():
        _wait_scatter(nb)
      _start_chunk(j + 1, nb)

    _wait_gather(b)

    def _scale16(k, _):
      w16 = plsc.bitcast(eps[b][2, pl.ds(k * 16, 16)], jnp.float32)
      c16 = eps[b][1, pl.ds(k * 16, 16)]
      plsc.addupdate_scatter(deg_v, [c16], w16)
      for j2 in range(16):
        e = k * 16 + j2
        ws = _splat(w16, j2)
        for f in range(IN_DIM // 16):
          sl = pl.ds(f * 16, 16)
          rows[b][e, sl] = rows[b][e, sl] * ws
      return 0
    lax.fori_loop(0, E_CHK // 16, _scale16, 0)

    pltpu.async_copy(rows[b], acc_sh.at[eps[b].at[1]], ssem[b], add=True)

  _start_chunk(0, 0)

  def _group(g, _):
    for t in range(NBUF):
      _do_chunk(g * NBUF + t, t)
    return 0
  lax.fori_loop(0, N_CHK // NBUF, _group, 0)

  for t in range(N_CHK % NBUF):
    _do_chunk((N_CHK // NBUF) * NBUF + t, t)

  for b in range(NBUF):
    _wait_scatter(b)

  pltpu.sync_copy(deg_v, outd_hbm.at[wid])

  plsc.subcore_barrier()

  def _writeout(t, _):
    blk = t * NS + s
    @pl.when(blk < N_IO_BLKS)
    def _():
      start = blk * IO_ROWS
      pltpu.sync_copy(acc_sh.at[pl.ds(start, IO_ROWS)], rows0)
      pltpu.sync_copy(rows0, out_hbm.at[c, pl.ds(start, IO_ROWS)])
    return 0
  lax.fori_loop(0, (N_IO_BLKS + NS - 1) // NS, _writeout, 0)


def _sc_aggregate(x, epack):
  mesh = plsc.VectorSubcoreMesh(core_axis_name="c", subcore_axis_name="s")
  return pl.kernel(
      _sc_body,
      out_type=(
          jax.ShapeDtypeStruct((NC, N_NODES, IN_DIM), jnp.float32),
          jax.ShapeDtypeStruct((NW, N_NODES), jnp.float32),
      ),
      mesh=mesh,
      compiler_params=pltpu.CompilerParams(needs_layout_passes=False),
      scratch_types=[
          pltpu.VMEM_SHARED((N_NODES, IN_DIM), jnp.float32),
          pltpu.VMEM((3, E_CHK), jnp.int32),
          pltpu.VMEM((3, E_CHK), jnp.int32),
          pltpu.VMEM((3, E_CHK), jnp.int32),
          pltpu.VMEM((E_CHK, IN_DIM), jnp.float32),
          pltpu.VMEM((E_CHK, IN_DIM), jnp.float32),
          pltpu.VMEM((E_CHK, IN_DIM), jnp.float32),
          pltpu.VMEM((N_NODES,), jnp.float32),
          pltpu.SemaphoreType.DMA,
          pltpu.SemaphoreType.DMA,
          pltpu.SemaphoreType.DMA,
          pltpu.SemaphoreType.DMA,
          pltpu.SemaphoreType.DMA,
          pltpu.SemaphoreType.DMA,
      ],
  )(x, epack)


ROW_BLK = 1000


def _tc_self_body(x_ref, wsT_ref, bs_ref, out_ref):
  out_ref[...] = jnp.dot(x_ref[...], wsT_ref[...],
                         preferred_element_type=jnp.float32) + bs_ref[...]


def _tc_self(x, W_self, b_self):
  return pl.pallas_call(
      _tc_self_body,
      grid=(N_NODES // ROW_BLK,),
      in_specs=[
          pl.BlockSpec((ROW_BLK, IN_DIM), lambda i: (i, 0)),
          pl.BlockSpec((IN_DIM, IN_DIM), lambda i: (0, 0)),
          pl.BlockSpec((1, IN_DIM), lambda i: (0, 0)),
      ],
      out_specs=pl.BlockSpec((ROW_BLK, IN_DIM), lambda i: (i, 0)),
      out_shape=jax.ShapeDtypeStruct((N_NODES, IN_DIM), jnp.float32),
  )(x, W_self.T, b_self[None, :])


def _tc_body(xs_ref, acc_ref, deg_ref, wnT_ref, bn_ref, out_ref):
  neigh = acc_ref[0] + acc_ref[1]
  deg = jnp.sum(deg_ref[0], axis=1, keepdims=True)
  xn = neigh / jnp.maximum(deg, 1.0)
  out_ref[...] = (
      xs_ref[...]
      + jnp.dot(xn, wnT_ref[...], preferred_element_type=jnp.float32)
      + bn_ref[...])


def _tc_combine(xs, acc, degT, W_neigh, b_neigh):
  grid = (N_NODES // ROW_BLK,)
  return pl.pallas_call(
      _tc_body,
      grid=grid,
      in_specs=[
          pl.BlockSpec((ROW_BLK, IN_DIM), lambda i: (i, 0)),
          pl.BlockSpec((NC, ROW_BLK, IN_DIM), lambda i: (0, i, 0)),
          pl.BlockSpec((1, ROW_BLK, NW), lambda i: (i, 0, 0)),
          pl.BlockSpec((IN_DIM, IN_DIM), lambda i: (0, 0)),
          pl.BlockSpec((1, IN_DIM), lambda i: (0, 0)),
      ],
      out_specs=pl.BlockSpec((ROW_BLK, IN_DIM), lambda i: (i, 0)),
      out_shape=jax.ShapeDtypeStruct((N_NODES, IN_DIM), jnp.float32),
  )(xs, acc, degT, W_neigh.T, b_neigh[None, :])


@jax.jit
def kernel(x, edge_index, edge_weight, W_self, b_self, W_neigh, b_neigh):
  row = edge_index[0].astype(jnp.int32).reshape(NW, N_CHK, 1, E_CHK)
  col = edge_index[1].astype(jnp.int32).reshape(NW, N_CHK, 1, E_CHK)
  wbits = lax.bitcast_convert_type(edge_weight, jnp.int32).reshape(
      NW, N_CHK, 1, E_CHK)
  epack = jnp.concatenate([row, col, wbits], axis=2)
  xs = _tc_self(x, W_self, b_self)
  acc, deg = _sc_aggregate(x, epack)
  degT = deg.reshape(NW, N_NODES // ROW_BLK, ROW_BLK).transpose(1, 2, 0)
  return _tc_combine(xs, acc, degT, W_neigh, b_neigh)


# speedup vs baseline: 1.4443x; 1.1291x over previous
import functools

import jax
import jax.numpy as jnp
from jax import lax
from jax.experimental import pallas as pl
from jax.experimental.pallas import tpu as pltpu
from jax.experimental.pallas import tpu_sc as plsc

N_NODES = 10000
IN_DIM = 128
N_EDGES = 320000
NC = 2
NS = 16
NW = NC * NS
E_PER_W = N_EDGES // NW
E_CHK = 80
N_CHK = E_PER_W // E_CHK
IO_ROWS = 80
N_IO_BLKS = N_NODES // IO_ROWS

_SPLAT_DN = lax.GatherDimensionNumbers(
    offset_dims=(), collapsed_slice_dims=(0,), start_index_map=(0,))


def _splat(v16, j):
  idx = jnp.full((16, 1), j, dtype=jnp.int32)
  return lax.gather(v16, idx, _SPLAT_DN, (1,),
                    mode=lax.GatherScatterMode.PROMISE_IN_BOUNDS)


NBUF = 3


def _sc_body(x_hbm, ep_hbm, out_hbm, outd_hbm,
             acc_sh, ep0, ep1, ep2, rows0, rows1, rows2,
             ci0, ci1, ci2, deg_v,
             e0, e1, e2, g0, g1, g2, s0, s1, s2):
  c = lax.axis_index("c")
  s = lax.axis_index("s")
  wid = c * NS + s
  eps = (ep0, ep1, ep2)
  rows = (rows0, rows1, rows2)
  cids = (ci0, ci1, ci2)
  esem = (e0, e1, e2)
  gsem = (g0, g1, g2)
  ssem = (s0, s1, s2)

  def _wait_gather(b):
    pltpu.make_async_copy(x_hbm.at[pl.ds(0, E_CHK)], rows[b],
                          gsem[b]).wait()

  def _wait_scatter(b):
    pltpu.make_async_copy(rows[b], acc_sh.at[pl.ds(0, E_CHK)],
                          ssem[b]).wait()

  def _copy_ep(j, b):
    pltpu.async_copy(ep_hbm.at[wid, j], eps[b], esem[b])

  def _wait_ep(b):
    pltpu.make_async_copy(ep_hbm.at[wid, 0], eps[b], esem[b]).wait()

  def _zero_row(i, _):
    for f in range(IN_DIM // 16):
      rows0[i, pl.ds(f * 16, 16)] = jnp.zeros((16,), jnp.float32)
    return 0
  lax.fori_loop(0, IO_ROWS, _zero_row, 0)

  def _zero_deg(i, _):
    deg_v[pl.ds(i * 16, 16)] = jnp.zeros((16,), jnp.float32)
    return 0
  lax.fori_loop(0, N_NODES // 16, _zero_deg, 0)

  def _zero_acc(t, _):
    blk = t * NS + s
    @pl.when(blk < N_IO_BLKS)
    def _():
      pltpu.sync_copy(rows0, acc_sh.at[pl.ds(blk * IO_ROWS, IO_ROWS)])
    return 0
  lax.fori_loop(0, (N_IO_BLKS + NS - 1) // NS, _zero_acc, 0)

  plsc.subcore_barrier()

  def _do_chunk(j, b):
    nb = (b + 1) % NBUF

    @pl.when(j + 2 < N_CHK)
    def _():
      _copy_ep(j + 2, (b + 2) % NBUF)

    @pl.when(j + 1 < N_CHK)
    def _():
      @pl.when(j + 1 >= NBUF)
      def _():
        _wait_scatter(nb)
      _wait_ep(nb)
      pltpu.async_copy(x_hbm.at[eps[nb].at[0]], rows[nb], gsem[nb])

    _wait_gather(b)

    def _scale16(k, _):
      w16 = plsc.bitcast(eps[b][2, pl.ds(k * 16, 16)], jnp.float32)
      c16 = eps[b][1, pl.ds(k * 16, 16)]
      cids[b][pl.ds(k * 16, 16)] = c16
      plsc.addupdate_scatter(deg_v, [c16], w16)
      for j2 in range(16):
        e = k * 16 + j2
        ws = _splat(w16, j2)
        for f in range(IN_DIM // 16):
          sl = pl.ds(f * 16, 16)
          rows[b][e, sl] = rows[b][e, sl] * ws
      return 0
    lax.fori_loop(0, E_CHK // 16, _scale16, 0)

    pltpu.async_copy(rows[b], acc_sh.at[cids[b]], ssem[b], add=True)

  _copy_ep(0, 0)
  _copy_ep(1, 1)
  _wait_ep(0)
  pltpu.async_copy(x_hbm.at[eps[0].at[0]], rows[0], gsem[0])

  def _group(g, _):
    for t in range(NBUF):
      _do_chunk(g * NBUF + t, t)
    return 0
  lax.fori_loop(0, N_CHK // NBUF, _group, 0)

  for t in range(N_CHK % NBUF):
    _do_chunk((N_CHK // NBUF) * NBUF + t, t)

  for b in range(NBUF):
    _wait_scatter(b)

  pltpu.sync_copy(deg_v, outd_hbm.at[wid])

  plsc.subcore_barrier()

  def _writeout(t, _):
    blk = t * NS + s
    @pl.when(blk < N_IO_BLKS)
    def _():
      start = blk * IO_ROWS
      pltpu.sync_copy(acc_sh.at[pl.ds(start, IO_ROWS)], rows0)
      pltpu.sync_copy(rows0, out_hbm.at[c, pl.ds(start, IO_ROWS)])
    return 0
  lax.fori_loop(0, (N_IO_BLKS + NS - 1) // NS, _writeout, 0)


def _sc_aggregate(x, epack):
  mesh = plsc.VectorSubcoreMesh(core_axis_name="c", subcore_axis_name="s")
  return pl.kernel(
      _sc_body,
      out_type=(
          jax.ShapeDtypeStruct((NC, N_NODES, IN_DIM), jnp.float32),
          jax.ShapeDtypeStruct((NW, N_NODES), jnp.float32),
      ),
      mesh=mesh,
      compiler_params=pltpu.CompilerParams(needs_layout_passes=False),
      scratch_types=[
          pltpu.VMEM_SHARED((N_NODES, IN_DIM), jnp.float32),
          pltpu.VMEM((3, E_CHK), jnp.int32),
          pltpu.VMEM((3, E_CHK), jnp.int32),
          pltpu.VMEM((3, E_CHK), jnp.int32),
          pltpu.VMEM((E_CHK, IN_DIM), jnp.float32),
          pltpu.VMEM((E_CHK, IN_DIM), jnp.float32),
          pltpu.VMEM((E_CHK, IN_DIM), jnp.float32),
          pltpu.VMEM((E_CHK,), jnp.int32),
          pltpu.VMEM((E_CHK,), jnp.int32),
          pltpu.VMEM((E_CHK,), jnp.int32),
          pltpu.VMEM((N_NODES,), jnp.float32),
          pltpu.SemaphoreType.DMA,
          pltpu.SemaphoreType.DMA,
          pltpu.SemaphoreType.DMA,
          pltpu.SemaphoreType.DMA,
          pltpu.SemaphoreType.DMA,
          pltpu.SemaphoreType.DMA,
          pltpu.SemaphoreType.DMA,
          pltpu.SemaphoreType.DMA,
          pltpu.SemaphoreType.DMA,
      ],
  )(x, epack)


ROW_BLK = 1000


def _tc_self_body(x_ref, wsT_ref, bs_ref, out_ref):
  out_ref[...] = jnp.dot(x_ref[...], wsT_ref[...],
                         preferred_element_type=jnp.float32) + bs_ref[...]


def _tc_self(x, W_self, b_self):
  return pl.pallas_call(
      _tc_self_body,
      grid=(N_NODES // ROW_BLK,),
      in_specs=[
          pl.BlockSpec((ROW_BLK, IN_DIM), lambda i: (i, 0)),
          pl.BlockSpec((IN_DIM, IN_DIM), lambda i: (0, 0)),
          pl.BlockSpec((1, IN_DIM), lambda i: (0, 0)),
      ],
      out_specs=pl.BlockSpec((ROW_BLK, IN_DIM), lambda i: (i, 0)),
      out_shape=jax.ShapeDtypeStruct((N_NODES, IN_DIM), jnp.float32),
  )(x, W_self.T, b_self[None, :])


def _tc_body(xs_ref, acc_ref, deg_ref, wnT_ref, bn_ref, out_ref):
  neigh = acc_ref[0] + acc_ref[1]
  deg = jnp.sum(deg_ref[0], axis=1, keepdims=True)
  xn = neigh / jnp.maximum(deg, 1.0)
  out_ref[...] = (
      xs_ref[...]
      + jnp.dot(xn, wnT_ref[...], preferred_element_type=jnp.float32)
      + bn_ref[...])


def _tc_combine(xs, acc, degT, W_neigh, b_neigh):
  grid = (N_NODES // ROW_BLK,)
  return pl.pallas_call(
      _tc_body,
      grid=grid,
      in_specs=[
          pl.BlockSpec((ROW_BLK, IN_DIM), lambda i: (i, 0)),
          pl.BlockSpec((NC, ROW_BLK, IN_DIM), lambda i: (0, i, 0)),
          pl.BlockSpec((1, ROW_BLK, NW), lambda i: (i, 0, 0)),
          pl.BlockSpec((IN_DIM, IN_DIM), lambda i: (0, 0)),
          pl.BlockSpec((1, IN_DIM), lambda i: (0, 0)),
      ],
      out_specs=pl.BlockSpec((ROW_BLK, IN_DIM), lambda i: (i, 0)),
      out_shape=jax.ShapeDtypeStruct((N_NODES, IN_DIM), jnp.float32),
  )(xs, acc, degT, W_neigh.T, b_neigh[None, :])


@jax.jit
def kernel(x, edge_index, edge_weight, W_self, b_self, W_neigh, b_neigh):
  row = edge_index[0].astype(jnp.int32).reshape(NW, N_CHK, 1, E_CHK)
  col = edge_index[1].astype(jnp.int32).reshape(NW, N_CHK, 1, E_CHK)
  wbits = lax.bitcast_convert_type(edge_weight, jnp.int32).reshape(
      NW, N_CHK, 1, E_CHK)
  epack = jnp.concatenate([row, col, wbits], axis=2)
  xs = _tc_self(x, W_self, b_self)
  acc, deg = _sc_aggregate(x, epack)
  degT = deg.reshape(NW, N_NODES // ROW_BLK, ROW_BLK).transpose(1, 2, 0)
  return _tc_combine(xs, acc, degT, W_neigh, b_neigh)
